# baseline (speedup 1.0000x reference)
import functools

import jax
import jax.numpy as jnp
from jax import lax
from jax.experimental import pallas as pl
from jax.experimental.pallas import tpu as pltpu
from jax.experimental.pallas import tpu_sc as plsc

N_EMB = 1000000
D = 64
B = 4096
L = 200

_info = plsc.get_sparse_core_info()
NC = _info.num_cores
NS = _info.num_subcores
NW = NC * NS
BT = B // NW
NBUF = 4


def _emb_body(xt_hbm, tok_hbm, pos_hbm, out_hbm, pos_v, idx_v, g_v, o_v, *sems):
    isems = sems[:NBUF]
    gsems = sems[NBUF:2 * NBUF]
    osems = sems[2 * NBUF:]
    wid = lax.axis_index("s") * NC + lax.axis_index("c")

    pltpu.sync_copy(pos_hbm.at[pl.ds(0, L)], pos_v)

    iota = lax.iota(jnp.int32, 16)
    rowsel = [bb * 16 + iota for bb in range(8)]

    def idx_desc(l, r):
        off = pl.multiple_of(l * B + wid * BT, 8)
        return pltpu.make_async_copy(
            xt_hbm.at[pl.ds(off, BT)], idx_v.at[r], isems[r])

    def gather_desc(l, r):
        del l
        return pltpu.make_async_copy(
            tok_hbm.at[idx_v.at[r]], g_v.at[r], gsems[r])

    def out_descs(l, r):
        return [
            pltpu.make_async_copy(
                o_v.at[r, pl.ds(dt * 8, 8)], out_hbm.at[l, dt, wid], osems[r])
            for dt in range(8)
        ]

    def compute(l, r):
        def dd_body(dd, carry):
            cidx = jnp.zeros((16,), jnp.int32) + dd
            lvec = jnp.zeros((16,), jnp.int32) + l
            psplat = plsc.load_gather(pos_v, [lvec, cidx])
            for bb in range(8):
                v = plsc.load_gather(g_v.at[r], [rowsel[bb], cidx]) + psplat
                o_v[r, dd, pl.ds(bb * 16, 16)] = v
            return carry
        lax.fori_loop(0, D, dd_body, 0)

    idx_desc(0, 0).start()
    idx_desc(1, 1).start()
    idx_desc(0, 0).wait()
    gather_desc(0, 0).start()

    def iter_body(l, cur, nxt, n2, *, osem_wait, g_next, i_next):
        if g_next:
            idx_desc(l + 1, nxt).wait()
            if osem_wait:
                for dsc in out_descs(l - 3, nxt):
                    dsc.wait()
            gather_desc(l + 1, nxt).start()
        if i_next:
            idx_desc(l + 2, n2).start()
        gather_desc(l, cur).wait()
        compute(l, cur)
        for dsc in out_descs(l, cur):
            dsc.start()

    for lh in range(3):
        iter_body(lh, lh % NBUF, (lh + 1) % NBUF, (lh + 2) % NBUF,
                  osem_wait=False, g_next=True, i_next=True)

    def group(g, carry):
        for j in range(4):
            l = 3 + g * 4 + j
            cur = (3 + j) % NBUF
            iter_body(l, cur, (cur + 1) % NBUF, (cur + 2) % NBUF,
                      osem_wait=True, g_next=True, i_next=True)
        return carry
    lax.fori_loop(0, 48, group, 0)

    for lt in range(195, 200):
        iter_body(lt, lt % NBUF, (lt + 1) % NBUF, (lt + 2) % NBUF,
                  osem_wait=True, g_next=(lt < 199), i_next=(lt < 198))

    for lt in range(196, 200):
        for dsc in out_descs(lt, lt % NBUF):
            dsc.wait()


_emb_call = functools.partial(
    pl.kernel,
    mesh=plsc.VectorSubcoreMesh(core_axis_name="c", subcore_axis_name="s"),
    out_type=jax.ShapeDtypeStruct((L, 8, NW, 8, 128), jnp.float32),
    scratch_types=[
        pltpu.VMEM((L, D), jnp.float32),
        pltpu.VMEM((NBUF, BT), jnp.int32),
        pltpu.VMEM((NBUF, BT, D), jnp.float32),
        pltpu.VMEM((NBUF, D, BT), jnp.float32),
    ] + [pltpu.SemaphoreType.DMA] * (3 * NBUF),
    compiler_params=pltpu.CompilerParams(
        use_tc_tiling_on_sc=False, needs_layout_passes=False),
)(_emb_body)


def kernel(x, token_table, pos_table):
    xt_flat = x.T.reshape(-1).astype(jnp.int32)
    out5 = _emb_call(xt_flat, token_table, pos_table)
    return out5.transpose(2, 4, 0, 1, 3).reshape(B, L, D)
